# baseline (speedup 1.0000x reference)
import functools

import jax
import jax.numpy as jnp
from jax import lax
from jax.experimental import pallas as pl
from jax.experimental.pallas import tpu as pltpu
from jax.experimental.pallas import tpu_sc as plsc

N = 10000
C = 128
K = 32
COUT = 128

NW = 32
NPAD = 10240
NBLK = 512
NBLOCKS = NPAD // NBLK
G = NBLK // 16
IBLK = K * NBLK
NQ = N // 16


def _sc_gather_max_build():
    mesh = plsc.VectorSubcoreMesh(core_axis_name="c", subcore_axis_name="s")

    @functools.partial(
        pl.kernel,
        out_type=jax.ShapeDtypeStruct((NW, 4 * NPAD), jnp.float32),
        mesh=mesh,
        compiler_params=pltpu.CompilerParams(needs_layout_passes=False),
        scratch_types=[
            pltpu.VMEM((N,), jnp.float32),
            pltpu.VMEM((N,), jnp.float32),
            pltpu.VMEM((N,), jnp.int32),
            pltpu.VMEM((N,), jnp.int32),
            pltpu.VMEM((2 * IBLK,), jnp.int32),
            pltpu.VMEM((4 * NPAD,), jnp.float32),
            pltpu.SemaphoreType.DMA((2,)),
        ],
    )
    def sc_kernel(x_hbm, idx_hbm, m_hbm,
                  s0_v, s1_v, p0_v, p1_v, idx_v, m_v, sems):
        t = lax.axis_index("s") * 2 + lax.axis_index("c")

        def issue(blk, bb):
            pltpu.async_copy(idx_hbm.at[blk],
                             idx_v.at[pl.ds(bb * IBLK, IBLK)],
                             sems.at[bb])

        def drain(bb):
            pltpu.make_async_copy(idx_hbm.at[0],
                                  idx_v.at[pl.ds(bb * IBLK, IBLK)],
                                  sems.at[bb]).wait()

        issue(0, 0)
        issue(1, 1)

        def build(p_v):
            def qbody(q, carry):
                qo = q * 16
                w = plsc.pack(s0_v[pl.ds(qo, 16)], s1_v[pl.ds(qo, 16)],
                              format=plsc.PackFormat.INTERLEAVED)
                p_v[pl.ds(qo, 16)] = plsc.bitcast(w, jnp.int32)
                return carry
            lax.fori_loop(0, NQ, qbody, 0)

        pltpu.sync_copy(x_hbm.at[4 * t], s0_v)
        pltpu.sync_copy(x_hbm.at[4 * t + 1], s1_v)
        build(p0_v)
        pltpu.sync_copy(x_hbm.at[4 * t + 2], s0_v)
        pltpu.sync_copy(x_hbm.at[4 * t + 3], s1_v)
        build(p1_v)

        lanes16 = lax.iota(jnp.int32, 16) * K

        def compute(blk, bb):
            base = bb * IBLK

            def gbody(g, carry):
                lanes_g = lanes16 + (base + g * (16 * K))
                acc0 = acc1 = None
                for k in range(K):
                    i_w = plsc.load_gather(idx_v, [lanes_g + k])
                    i_s = i_w & 0xFFFF
                    i_d = lax.shift_right_logical(i_w, 16)
                    s0 = plsc.load_gather(p0_v, [i_s])
                    d0 = plsc.load_gather(p0_v, [i_d])
                    s1 = plsc.load_gather(p1_v, [i_s])
                    d1 = plsc.load_gather(p1_v, [i_d])
                    v0 = (plsc.bitcast(s0, jnp.bfloat16)
                          - plsc.bitcast(d0, jnp.bfloat16))
                    v1 = (plsc.bitcast(s1, jnp.bfloat16)
                          - plsc.bitcast(d1, jnp.bfloat16))
                    if acc0 is None:
                        acc0, acc1 = v0, v1
                    else:
                        acc0 = jnp.maximum(acc0, v0)
                        acc1 = jnp.maximum(acc1, v1)
                ca, cb = plsc.unpack(acc0,
                                     format=plsc.PackFormat.INTERLEAVED)
                cc, cd = plsc.unpack(acc1,
                                     format=plsc.PackFormat.INTERLEAVED)
                noff = blk * NBLK + g * 16
                m_v[pl.ds(noff, 16)] = ca
                m_v[pl.ds(NPAD + noff, 16)] = cb
                m_v[pl.ds(2 * NPAD + noff, 16)] = cc
                m_v[pl.ds(3 * NPAD + noff, 16)] = cd
                return carry

            lax.fori_loop(0, G, gbody, 0)

        def body(i, carry):
            b0 = i * 2
            drain(0)
            compute(b0, 0)
            issue(b0 + 2, 0)
            drain(1)
            compute(b0 + 1, 1)
            issue(b0 + 3, 1)
            return carry

        lax.fori_loop(0, NBLOCKS // 2, body, 0)
        drain(0)
        drain(1)
        pltpu.sync_copy(m_v, m_hbm.at[t])

    return sc_kernel


_sc_gather_max = _sc_gather_max_build()


def _tc_body(x_ref, m_ref, w1_ref, w2_ref, b_ref, o_ref):
    acc = lax.dot_general(w1_ref[...], x_ref[...],
                          (((1,), (0,)), ((), ())),
                          preferred_element_type=jnp.float32)
    acc = acc + lax.dot_general(w2_ref[...], m_ref[:, 0:N],
                                (((1,), (0,)), ((), ())),
                                preferred_element_type=jnp.float32)
    o_ref[...] = jnp.maximum(acc + b_ref[...], 0.0)


def _tc_matmul(x2d, m2d, w1, w2, b2):
    return pl.pallas_call(
        _tc_body,
        out_shape=jax.ShapeDtypeStruct((COUT, N), jnp.float32),
    )(x2d, m2d, w1, w2, b2)


def kernel(x, edge_index, W, bconv):
    x2d = x.reshape(C, N)
    idx = edge_index.reshape(2, N, K).astype(jnp.int32)
    idxp = idx[0] | (idx[1] << 16)
    idxp = jnp.pad(idxp, ((0, NPAD - N), (0, 0)))
    idxp = idxp.reshape(NBLOCKS, IBLK)
    idxp = jnp.pad(idxp, ((0, 2), (0, 0)))
    mp = _sc_gather_max(x2d, idxp)
    m2d = mp.reshape(C, NPAD)
    w1 = W[:, :C]
    w2 = W[:, C:]
    b2 = bconv.reshape(COUT, 1)
    out = _tc_matmul(x2d, m2d, w1, w2, b2)
    return out.reshape(1, COUT, N, 1)


# speedup vs baseline: 2.1403x; 2.1403x over previous
import functools

import jax
import jax.numpy as jnp
from jax import lax
from jax.experimental import pallas as pl
from jax.experimental.pallas import tpu as pltpu
from jax.experimental.pallas import tpu_sc as plsc

N = 10000
C = 128
K = 32
COUT = 128

NW = 32
NPAD = 10240
NBLK = 512
NBLOCKS = NPAD // NBLK
G = NBLK // 16
IBLK = K * NBLK
NQ = N // 16


def _sc_gather_max_build():
    mesh = plsc.VectorSubcoreMesh(core_axis_name="c", subcore_axis_name="s")

    @functools.partial(
        pl.kernel,
        out_type=jax.ShapeDtypeStruct((NW, 4 * NPAD), jnp.float32),
        mesh=mesh,
        compiler_params=pltpu.CompilerParams(needs_layout_passes=False),
        scratch_types=[
            pltpu.VMEM((N,), jnp.float32),
            pltpu.VMEM((N,), jnp.float32),
            pltpu.VMEM((N,), jnp.int32),
            pltpu.VMEM((N,), jnp.int32),
            pltpu.VMEM((2 * IBLK,), jnp.int32),
            pltpu.VMEM((4 * NPAD,), jnp.float32),
            pltpu.SemaphoreType.DMA((2,)),
        ],
    )
    def sc_kernel(x_hbm, idx_hbm, m_hbm,
                  s0_v, s1_v, p0_v, p1_v, idx_v, m_v, sems):
        t = lax.axis_index("s") * 2 + lax.axis_index("c")

        def issue(blk, bb):
            pltpu.async_copy(idx_hbm.at[blk],
                             idx_v.at[pl.ds(bb * IBLK, IBLK)],
                             sems.at[bb])

        def drain(bb):
            pltpu.make_async_copy(idx_hbm.at[0],
                                  idx_v.at[pl.ds(bb * IBLK, IBLK)],
                                  sems.at[bb]).wait()

        issue(0, 0)
        issue(1, 1)

        def build(p_v):
            def qbody(q, carry):
                qo = q * 16
                w = plsc.pack(s0_v[pl.ds(qo, 16)], s1_v[pl.ds(qo, 16)],
                              format=plsc.PackFormat.INTERLEAVED)
                p_v[pl.ds(qo, 16)] = plsc.bitcast(w, jnp.int32)
                return carry
            lax.fori_loop(0, NQ, qbody, 0)

        pltpu.sync_copy(x_hbm.at[4 * t], s0_v)
        pltpu.sync_copy(x_hbm.at[4 * t + 1], s1_v)
        build(p0_v)
        pltpu.sync_copy(x_hbm.at[4 * t + 2], s0_v)
        pltpu.sync_copy(x_hbm.at[4 * t + 3], s1_v)
        build(p1_v)

        def compute(blk, bb):
            base = bb * IBLK

            def gbody(g, carry):
                goff = g * 16
                acc0 = acc1 = None
                for k in range(K):
                    i_w = idx_v[pl.ds(base + k * NBLK + goff, 16)]
                    i_s = i_w & 0xFFFF
                    i_d = lax.shift_right_logical(i_w, 16)
                    s0 = plsc.load_gather(p0_v, [i_s])
                    d0 = plsc.load_gather(p0_v, [i_d])
                    s1 = plsc.load_gather(p1_v, [i_s])
                    d1 = plsc.load_gather(p1_v, [i_d])
                    v0 = (plsc.bitcast(s0, jnp.bfloat16)
                          - plsc.bitcast(d0, jnp.bfloat16))
                    v1 = (plsc.bitcast(s1, jnp.bfloat16)
                          - plsc.bitcast(d1, jnp.bfloat16))
                    if acc0 is None:
                        acc0, acc1 = v0, v1
                    else:
                        acc0 = jnp.maximum(acc0, v0)
                        acc1 = jnp.maximum(acc1, v1)
                ca, cb = plsc.unpack(acc0,
                                     format=plsc.PackFormat.INTERLEAVED)
                cc, cd = plsc.unpack(acc1,
                                     format=plsc.PackFormat.INTERLEAVED)
                noff = blk * NBLK + goff
                m_v[pl.ds(noff, 16)] = ca
                m_v[pl.ds(NPAD + noff, 16)] = cb
                m_v[pl.ds(2 * NPAD + noff, 16)] = cc
                m_v[pl.ds(3 * NPAD + noff, 16)] = cd
                return carry

            lax.fori_loop(0, G, gbody, 0)

        def body(i, carry):
            b0 = i * 2
            drain(0)
            compute(b0, 0)
            issue(b0 + 2, 0)
            drain(1)
            compute(b0 + 1, 1)
            issue(b0 + 3, 1)
            return carry

        lax.fori_loop(0, NBLOCKS // 2, body, 0)
        drain(0)
        drain(1)
        pltpu.sync_copy(m_v, m_hbm.at[t])

    return sc_kernel


_sc_gather_max = _sc_gather_max_build()


def _tc_body(x_ref, m_ref, w1_ref, w2_ref, b_ref, o_ref):
    acc = lax.dot_general(w1_ref[...], x_ref[...],
                          (((1,), (0,)), ((), ())),
                          preferred_element_type=jnp.float32)
    acc = acc + lax.dot_general(w2_ref[...], m_ref[:, 0:N],
                                (((1,), (0,)), ((), ())),
                                preferred_element_type=jnp.float32)
    o_ref[...] = jnp.maximum(acc + b_ref[...], 0.0)


def _tc_matmul(x2d, m2d, w1, w2, b2):
    return pl.pallas_call(
        _tc_body,
        out_shape=jax.ShapeDtypeStruct((COUT, N), jnp.float32),
    )(x2d, m2d, w1, w2, b2)


def kernel(x, edge_index, W, bconv):
    x2d = x.reshape(C, N)
    idx = edge_index.reshape(2, N, K).astype(jnp.int32)
    idxp = idx[0] | (idx[1] << 16)
    idxp = jnp.pad(idxp, ((0, NPAD - N), (0, 0)))
    idxp = idxp.T.reshape(K, NBLOCKS, NBLK)
    idxp = idxp.transpose(1, 0, 2).reshape(NBLOCKS, IBLK)
    idxp = jnp.pad(idxp, ((0, 2), (0, 0)))
    mp = _sc_gather_max(x2d, idxp)
    m2d = mp.reshape(C, NPAD)
    w1 = W[:, :C]
    w2 = W[:, C:]
    b2 = bconv.reshape(COUT, 1)
    out = _tc_matmul(x2d, m2d, w1, w2, b2)
    return out.reshape(1, COUT, N, 1)


# speedup vs baseline: 2.2005x; 1.0282x over previous
import functools

import jax
import jax.numpy as jnp
from jax import lax
from jax.experimental import pallas as pl
from jax.experimental.pallas import tpu as pltpu
from jax.experimental.pallas import tpu_sc as plsc

N = 10000
C = 128
K = 32
COUT = 128

NW = 32
NPAD = 10240
NBLK = 512
NBLOCKS = NPAD // NBLK
G = NBLK // 16
IBLK = K * NBLK
NQ = N // 16


def _sc_gather_max_build():
    mesh = plsc.VectorSubcoreMesh(core_axis_name="c", subcore_axis_name="s")

    @functools.partial(
        pl.kernel,
        out_type=jax.ShapeDtypeStruct((NW, 4 * NPAD), jnp.float32),
        mesh=mesh,
        compiler_params=pltpu.CompilerParams(needs_layout_passes=False),
        scratch_types=[
            pltpu.VMEM((N,), jnp.float32),
            pltpu.VMEM((N,), jnp.float32),
            pltpu.VMEM((N,), jnp.int32),
            pltpu.VMEM((N,), jnp.int32),
            pltpu.VMEM((2 * IBLK,), jnp.int32),
            pltpu.VMEM((4 * NPAD,), jnp.float32),
            pltpu.SemaphoreType.DMA((2,)),
        ],
    )
    def sc_kernel(x_hbm, idx_hbm, m_hbm,
                  s0_v, s1_v, p0_v, p1_v, idx_v, m_v, sems):
        t = lax.axis_index("s") * 2 + lax.axis_index("c")

        def issue(blk, bb):
            pltpu.async_copy(idx_hbm.at[blk],
                             idx_v.at[pl.ds(bb * IBLK, IBLK)],
                             sems.at[bb])

        def drain(bb):
            pltpu.make_async_copy(idx_hbm.at[0],
                                  idx_v.at[pl.ds(bb * IBLK, IBLK)],
                                  sems.at[bb]).wait()

        issue(0, 0)
        issue(1, 1)

        def build(p_v):
            def qbody(q, carry):
                qo = q * 16
                w = plsc.pack(s0_v[pl.ds(qo, 16)], s1_v[pl.ds(qo, 16)],
                              format=plsc.PackFormat.INTERLEAVED)
                p_v[pl.ds(qo, 16)] = plsc.bitcast(w, jnp.int32)
                return carry
            lax.fori_loop(0, NQ, qbody, 0)

        pltpu.sync_copy(x_hbm.at[4 * t], s0_v)
        pltpu.sync_copy(x_hbm.at[4 * t + 1], s1_v)
        build(p0_v)
        pltpu.sync_copy(x_hbm.at[4 * t + 2], s0_v)
        pltpu.sync_copy(x_hbm.at[4 * t + 3], s1_v)
        build(p1_v)

        def compute(blk, bb):
            base = bb * IBLK

            def gbody(g, carry):
                goff = g * 16
                acc0 = acc1 = None
                for k in range(K):
                    i_w = idx_v[pl.ds(base + k * NBLK + goff, 16)]
                    i_s = i_w & 0xFFFF
                    i_d = lax.shift_right_logical(i_w, 16)
                    s0 = plsc.load_gather(p0_v, [i_s])
                    d0 = plsc.load_gather(p0_v, [i_d])
                    s1 = plsc.load_gather(p1_v, [i_s])
                    d1 = plsc.load_gather(p1_v, [i_d])
                    v0 = (plsc.bitcast(s0, jnp.bfloat16)
                          - plsc.bitcast(d0, jnp.bfloat16))
                    v1 = (plsc.bitcast(s1, jnp.bfloat16)
                          - plsc.bitcast(d1, jnp.bfloat16))
                    if acc0 is None:
                        acc0, acc1 = v0, v1
                    else:
                        acc0 = jnp.maximum(acc0, v0)
                        acc1 = jnp.maximum(acc1, v1)
                ca, cb = plsc.unpack(acc0,
                                     format=plsc.PackFormat.INTERLEAVED)
                cc, cd = plsc.unpack(acc1,
                                     format=plsc.PackFormat.INTERLEAVED)
                noff = blk * NBLK + goff
                m_v[pl.ds(noff, 16)] = ca
                m_v[pl.ds(NPAD + noff, 16)] = cb
                m_v[pl.ds(2 * NPAD + noff, 16)] = cc
                m_v[pl.ds(3 * NPAD + noff, 16)] = cd
                return carry

            lax.fori_loop(0, G, gbody, 0)

        def body(i, carry):
            b0 = i * 2
            drain(0)
            compute(b0, 0)
            issue(b0 + 2, 0)
            drain(1)
            compute(b0 + 1, 1)
            issue(b0 + 3, 1)
            return carry

        lax.fori_loop(0, NBLOCKS // 2, body, 0)
        drain(0)
        drain(1)
        pltpu.sync_copy(m_v, m_hbm.at[t])

    return sc_kernel


_sc_gather_max = _sc_gather_max_build()


def _tc_body(x_ref, m_ref, w1_ref, w2s_ref, b_ref, o_ref):
    acc = lax.dot_general(w1_ref[...], x_ref[...],
                          (((1,), (0,)), ((), ())),
                          preferred_element_type=jnp.float32)
    for j in range(4):
        acc = acc + lax.dot_general(
            w2s_ref[j], m_ref[:, j * NPAD:j * NPAD + N],
            (((1,), (0,)), ((), ())),
            preferred_element_type=jnp.float32)
    o_ref[...] = jnp.maximum(acc + b_ref[...], 0.0)


def _tc_matmul(x2d, mp, w1, w2s, b2):
    return pl.pallas_call(
        _tc_body,
        out_shape=jax.ShapeDtypeStruct((COUT, N), jnp.float32),
    )(x2d, mp, w1, w2s, b2)


def kernel(x, edge_index, W, bconv):
    x2d = x.reshape(C, N)
    idx = edge_index.reshape(2, N, K).astype(jnp.int32)
    idxp = idx[0] | (idx[1] << 16)
    idxp = jnp.pad(idxp, ((0, NPAD - N), (0, 0)))
    idxp = idxp.T.reshape(K, NBLOCKS, NBLK)
    idxp = idxp.transpose(1, 0, 2).reshape(NBLOCKS, IBLK)
    idxp = jnp.pad(idxp, ((0, 2), (0, 0)))
    mp = _sc_gather_max(x2d, idxp)
    w1 = W[:, :C]
    w2 = W[:, C:]
    w2s = w2.reshape(COUT, NW, 4).transpose(2, 0, 1)
    b2 = bconv.reshape(COUT, 1)
    out = _tc_matmul(x2d, mp, w1, w2s, b2)
    return out.reshape(1, COUT, N, 1)


# speedup vs baseline: 2.2029x; 1.0011x over previous
import functools

import jax
import jax.numpy as jnp
from jax import lax
from jax.experimental import pallas as pl
from jax.experimental.pallas import tpu as pltpu
from jax.experimental.pallas import tpu_sc as plsc

N = 10000
C = 128
K = 32
COUT = 128

NW = 32
NPAD = 10240
NBLK = 512
NBLOCKS = NPAD // NBLK
G = NBLK // 16
IBLK = K * NBLK
NQ = N // 16


def _sc_gather_max_build():
    mesh = plsc.VectorSubcoreMesh(core_axis_name="c", subcore_axis_name="s")

    @functools.partial(
        pl.kernel,
        out_type=jax.ShapeDtypeStruct((NW, 4 * NPAD), jnp.float32),
        mesh=mesh,
        compiler_params=pltpu.CompilerParams(needs_layout_passes=False),
        scratch_types=[
            pltpu.VMEM((N,), jnp.float32),
            pltpu.VMEM((N,), jnp.float32),
            pltpu.VMEM((N,), jnp.int32),
            pltpu.VMEM((N,), jnp.int32),
            pltpu.VMEM((2 * IBLK,), jnp.int32),
            pltpu.VMEM((4 * NPAD,), jnp.float32),
            pltpu.SemaphoreType.DMA((2,)),
        ],
    )
    def sc_kernel(x_hbm, idx_hbm, m_hbm,
                  s0_v, s1_v, p0_v, p1_v, idx_v, m_v, sems):
        t = lax.axis_index("s") * 2 + lax.axis_index("c")

        def issue(blk, bb):
            pltpu.async_copy(idx_hbm.at[blk],
                             idx_v.at[pl.ds(bb * IBLK, IBLK)],
                             sems.at[bb])

        def drain(bb):
            pltpu.make_async_copy(idx_hbm.at[0],
                                  idx_v.at[pl.ds(bb * IBLK, IBLK)],
                                  sems.at[bb]).wait()

        issue(0, 0)
        issue(1, 1)

        def build(p_v):
            def qbody(q, carry):
                qo = q * 16
                w = plsc.pack(s0_v[pl.ds(qo, 16)], s1_v[pl.ds(qo, 16)],
                              format=plsc.PackFormat.INTERLEAVED)
                p_v[pl.ds(qo, 16)] = plsc.bitcast(w, jnp.int32)
                return carry
            lax.fori_loop(0, NQ, qbody, 0)

        pltpu.sync_copy(x_hbm.at[4 * t], s0_v)
        pltpu.sync_copy(x_hbm.at[4 * t + 1], s1_v)
        build(p0_v)
        pltpu.sync_copy(x_hbm.at[4 * t + 2], s0_v)
        pltpu.sync_copy(x_hbm.at[4 * t + 3], s1_v)
        build(p1_v)

        def compute(blk, bb):
            base = bb * IBLK

            def gbody(g, carry):
                goff = g * 16
                acc0 = acc1 = None
                for k in range(K):
                    i_w = idx_v[pl.ds(base + k * NBLK + goff, 16)]
                    i_s = i_w & 0xFFFF
                    i_d = lax.shift_right_logical(i_w, 16)
                    s0 = plsc.load_gather(p0_v, [i_s])
                    d0 = plsc.load_gather(p0_v, [i_d])
                    s1 = plsc.load_gather(p1_v, [i_s])
                    d1 = plsc.load_gather(p1_v, [i_d])
                    v0 = (plsc.bitcast(s0, jnp.bfloat16)
                          - plsc.bitcast(d0, jnp.bfloat16))
                    v1 = (plsc.bitcast(s1, jnp.bfloat16)
                          - plsc.bitcast(d1, jnp.bfloat16))
                    if acc0 is None:
                        acc0, acc1 = v0, v1
                    else:
                        acc0 = jnp.maximum(acc0, v0)
                        acc1 = jnp.maximum(acc1, v1)
                ca, cb = plsc.unpack(acc0,
                                     format=plsc.PackFormat.INTERLEAVED)
                cc, cd = plsc.unpack(acc1,
                                     format=plsc.PackFormat.INTERLEAVED)
                noff = blk * NBLK + goff
                m_v[pl.ds(noff, 16)] = ca
                m_v[pl.ds(NPAD + noff, 16)] = cb
                m_v[pl.ds(2 * NPAD + noff, 16)] = cc
                m_v[pl.ds(3 * NPAD + noff, 16)] = cd
                return carry

            lax.fori_loop(0, G, gbody, 0)

        def body(i, carry):
            b0 = i * 2
            drain(0)
            compute(b0, 0)
            issue(b0 + 2, 0)
            drain(1)
            compute(b0 + 1, 1)
            issue(b0 + 3, 1)
            return carry

        lax.fori_loop(0, NBLOCKS // 2, body, 0)
        drain(0)
        drain(1)
        pltpu.sync_copy(m_v, m_hbm.at[t])

    return sc_kernel


_sc_gather_max = _sc_gather_max_build()


def _tc_body(x_ref, m_ref, w1_ref, w2s_ref, b_ref, o_ref):
    acc = lax.dot_general(w1_ref[...], x_ref[...],
                          (((1,), (0,)), ((), ())),
                          preferred_element_type=jnp.float32)
    for j in range(4):
        acc = acc + lax.dot_general(
            w2s_ref[j], m_ref[:, j * NPAD:j * NPAD + N],
            (((1,), (0,)), ((), ())),
            preferred_element_type=jnp.float32)
    o_ref[...] = jnp.maximum(acc + b_ref[...], 0.0)


def _tc_matmul(x2d, mp, w1, w2s, b2):
    return pl.pallas_call(
        _tc_body,
        out_shape=jax.ShapeDtypeStruct((COUT, N), jnp.float32),
    )(x2d, mp, w1, w2s, b2)


def kernel(x, edge_index, W, bconv):
    x2d = x.reshape(C, N)
    idx = edge_index.reshape(2, N, K).astype(jnp.int32)
    idxp = idx[0].T | (idx[1].T << 16)
    idxp = jnp.pad(idxp, ((0, 0), (0, NPAD - N)))
    idxp = idxp.reshape(K, NBLOCKS, NBLK)
    idxp = idxp.transpose(1, 0, 2).reshape(NBLOCKS, IBLK)
    idxp = jnp.pad(idxp, ((0, 2), (0, 0)))
    mp = _sc_gather_max(x2d, idxp)
    w1 = W[:, :C]
    w2 = W[:, C:]
    w2s = w2.reshape(COUT, NW, 4).transpose(2, 0, 1)
    b2 = bconv.reshape(COUT, 1)
    out = _tc_matmul(x2d, mp, w1, w2s, b2)
    return out.reshape(1, COUT, N, 1)
